# baseline (speedup 1.0000x reference)
import functools

import jax
import jax.numpy as jnp
from jax import lax
from jax.experimental import pallas as pl
from jax.experimental.pallas import tpu as pltpu
from jax.experimental.pallas import tpu_sc as plsc

BATCH = 16384
N_FIELDS = 26
FEATURE_DIM = 1000000
NC = 2
NS = 16
NW = NC * NS
B_PER_W = BATCH // NW
CHUNK = B_PER_W * N_FIELDS
LANES = 16
N_VECS = B_PER_W // LANES
FIELD_GROUPS = ((0, 7), (7, 14), (14, 20), (20, 26))


@functools.partial(
    pl.kernel,
    out_type=jax.ShapeDtypeStruct((BATCH,), jnp.float32),
    mesh=plsc.VectorSubcoreMesh(core_axis_name="c", subcore_axis_name="s"),
    compiler_params=pltpu.CompilerParams(needs_layout_passes=False,
                                         use_tc_tiling_on_sc=True),
    scratch_types=[
        pltpu.VMEM((CHUNK,), jnp.int32),
        pltpu.VMEM((CHUNK,), jnp.float32),
        pltpu.VMEM((CHUNK,), jnp.float32),
        pltpu.VMEM((B_PER_W,), jnp.float32),
        pltpu.SemaphoreType.DMA,
        pltpu.SemaphoreType.DMA,
        pltpu.SemaphoreType.DMA,
        pltpu.SemaphoreType.DMA,
    ],
)
def _sc_linear(idx_hbm, vals_hbm, w_hbm, out_hbm, idx_v, rows_v, vals_v,
               out_v, sem0, sem1, sem2, sem3):
    sems = (sem0, sem1, sem2, sem3)
    w = lax.axis_index("c") * NS + lax.axis_index("s")
    gathers = []
    for k, (lo, hi) in enumerate(FIELD_GROUPS):
        sl = pl.ds(lo * B_PER_W, (hi - lo) * B_PER_W)
        pltpu.sync_copy(idx_hbm.at[w, sl], idx_v.at[sl])
        gathers.append(
            pltpu.async_copy(w_hbm.at[0].at[idx_v.at[sl]], rows_v.at[sl],
                             sems[k]))
    pltpu.sync_copy(vals_hbm.at[w], vals_v)

    for k, (lo, hi) in enumerate(FIELD_GROUPS):
        gathers[k].wait()

        def body(s, carry, lo=lo, hi=hi, first=(k == 0)):
            base = s * LANES
            acc = (jnp.zeros((LANES,), jnp.float32) if first
                   else out_v[pl.ds(base, LANES)])
            for j in range(lo, hi):
                off = pl.ds(j * B_PER_W + base, LANES)
                acc = acc + rows_v[off] * vals_v[off]
            out_v[pl.ds(base, LANES)] = acc
            return carry

        lax.fori_loop(0, N_VECS, body, 0)
    pltpu.sync_copy(out_v, out_hbm.at[pl.ds(w * B_PER_W, B_PER_W)])


def _transpose_body(i_ref, v_ref, it_ref, vt_ref):
    it_ref[...] = i_ref[...].swapaxes(1, 2)
    vt_ref[...] = v_ref[...].swapaxes(1, 2)


_tc_transpose = pl.pallas_call(
    _transpose_body,
    grid=(NW,),
    in_specs=[
        pl.BlockSpec((1, B_PER_W, N_FIELDS), lambda i: (i, 0, 0)),
        pl.BlockSpec((1, B_PER_W, N_FIELDS), lambda i: (i, 0, 0)),
    ],
    out_specs=[
        pl.BlockSpec((1, N_FIELDS, B_PER_W), lambda i: (i, 0, 0)),
        pl.BlockSpec((1, N_FIELDS, B_PER_W), lambda i: (i, 0, 0)),
    ],
    out_shape=[
        jax.ShapeDtypeStruct((NW, N_FIELDS, B_PER_W), jnp.int32),
        jax.ShapeDtypeStruct((NW, N_FIELDS, B_PER_W), jnp.float32),
    ],
)


def kernel(feature_idx, feature_vals, W, b):
    idx3 = feature_idx.astype(jnp.int32).reshape(NW, B_PER_W, N_FIELDS)
    vals3 = feature_vals.reshape(NW, B_PER_W, N_FIELDS)
    idx_t, vals_t = _tc_transpose(idx3, vals3)
    out = _sc_linear(idx_t.reshape(NW, CHUNK), vals_t.reshape(NW, CHUNK),
                     W.T)
    return out.reshape(BATCH, 1) + b


# speedup vs baseline: 1.7123x; 1.7123x over previous
import functools

import jax
import jax.numpy as jnp
from jax import lax
from jax.experimental import pallas as pl
from jax.experimental.pallas import tpu as pltpu
from jax.experimental.pallas import tpu_sc as plsc

BATCH = 16384
N_FIELDS = 26
FEATURE_DIM = 1000000
NC = 2
NS = 16
NW = NC * NS
B_PER_W = BATCH // NW
CHUNK = B_PER_W * N_FIELDS
LANES = 16
N_VECS = B_PER_W // LANES
FIELD_GROUPS = ((0, 7), (7, 14), (14, 20), (20, 26))


@functools.partial(
    pl.kernel,
    out_type=jax.ShapeDtypeStruct((BATCH,), jnp.float32),
    mesh=plsc.VectorSubcoreMesh(core_axis_name="c", subcore_axis_name="s"),
    compiler_params=pltpu.CompilerParams(needs_layout_passes=False,
                                         use_tc_tiling_on_sc=True),
    scratch_types=[
        pltpu.VMEM((CHUNK,), jnp.int32),
        pltpu.VMEM((CHUNK,), jnp.float32),
        pltpu.VMEM((CHUNK,), jnp.float32),
        pltpu.VMEM((B_PER_W,), jnp.float32),
        pltpu.SemaphoreType.DMA,
        pltpu.SemaphoreType.DMA,
        pltpu.SemaphoreType.DMA,
        pltpu.SemaphoreType.DMA,
    ],
)
def _sc_linear(idx_hbm, vals_hbm, w_hbm, out_hbm, idx_v, rows_v, vals_v,
               out_v, sem0, sem1, sem2, sem3):
    sems = (sem0, sem1, sem2, sem3)
    w = lax.axis_index("c") * NS + lax.axis_index("s")
    gathers = []
    for k, (lo, hi) in enumerate(FIELD_GROUPS):
        sl = pl.ds(lo * B_PER_W, (hi - lo) * B_PER_W)
        pltpu.sync_copy(idx_hbm.at[w, sl], idx_v.at[sl])
        gathers.append(
            pltpu.async_copy(w_hbm.at[0].at[idx_v.at[sl]], rows_v.at[sl],
                             sems[k]))
    pltpu.sync_copy(vals_hbm.at[w], vals_v)

    for k, (lo, hi) in enumerate(FIELD_GROUPS):
        gathers[k].wait()

        def body(s, carry, lo=lo, hi=hi, first=(k == 0)):
            base = s * LANES
            acc0 = (jnp.zeros((LANES,), jnp.float32) if first
                    else out_v[pl.ds(base, LANES)])

            def jbody(j, acc):
                off = pl.ds(j * B_PER_W + base, LANES)
                return acc + rows_v[off] * vals_v[off]

            out_v[pl.ds(base, LANES)] = lax.fori_loop(lo, hi, jbody, acc0)
            return carry

        lax.fori_loop(0, N_VECS, body, 0)
    pltpu.sync_copy(out_v, out_hbm.at[pl.ds(w * B_PER_W, B_PER_W)])


def kernel(feature_idx, feature_vals, W, b):
    idx = (feature_idx.astype(jnp.int32)
           .reshape(NW, B_PER_W, N_FIELDS).transpose(0, 2, 1)
           .reshape(NW, CHUNK))
    vals = (feature_vals.reshape(NW, B_PER_W, N_FIELDS).transpose(0, 2, 1)
            .reshape(NW, CHUNK))
    out = _sc_linear(idx, vals, W.T)
    return out.reshape(BATCH, 1) + b
